# baseline (speedup 1.0000x reference)
import functools

import jax
import jax.numpy as jnp
from jax import lax
from jax.experimental import pallas as pl
from jax.experimental.pallas import tpu as pltpu
from jax.experimental.pallas import tpu_sc as plsc

D = 32
L = 16
TW = 128

SC_FRAC_NUM, SC_FRAC_DEN = 1, 2


@functools.lru_cache(maxsize=None)
def _build_sc(Bs):
    info = plsc.get_sparse_core_info()
    NC, NS = info.num_cores, info.num_subcores
    NW = NC * NS
    bpw = Bs // NW
    NG = bpw // L

    mesh = plsc.VectorSubcoreMesh(core_axis_name="c", subcore_axis_name="s")

    @functools.partial(
        pl.kernel,
        mesh=mesh,
        out_type=jax.ShapeDtypeStruct((Bs,), jnp.float32),
        compiler_params=pltpu.CompilerParams(
            needs_layout_passes=False, disable_bounds_checks=True),
        scratch_types=[
            pltpu.VMEM((bpw,), jnp.int32),
            pltpu.VMEM((bpw,), jnp.int32),
            pltpu.VMEM((L, D, TW), jnp.float32),
            pltpu.VMEM((D, L), jnp.float32),
            pltpu.VMEM((D,), jnp.float32),
            pltpu.VMEM((L,), jnp.float32),
            pltpu.VMEM((bpw,), jnp.float32),
            pltpu.SemaphoreType.DMA,
        ],
    )
    def sc_kernel(uidx_h, iidx_h, utabT_h, itabT_h, w_h, b_h, out_h,
                  uixv, iixv, blk, stage, wv, bv, outv, sem):
        wid = lax.axis_index("s") * NC + lax.axis_index("c")
        base = wid * bpw

        pltpu.sync_copy(uidx_h.at[pl.ds(base, bpw)], uixv)
        pltpu.sync_copy(iidx_h.at[pl.ds(base, bpw)], iixv)
        pltpu.sync_copy(w_h, wv)
        pltpu.sync_copy(b_h, bv)

        w_lo = wv[pl.ds(0, L)]
        w_hi = wv[pl.ds(L, L)]
        bvec = bv[...]
        lane = lax.iota(jnp.int32, L)

        def fetch_blocks(tab_h, cs):
            for j in range(L):
                off = pl.multiple_of(cs[j], TW)
                pltpu.async_copy(tab_h.at[:, pl.ds(off, TW)], blk.at[j], sem)
            for j in range(L):
                pltpu.make_async_copy(
                    tab_h.at[:, pl.ds(0, TW)], blk.at[j], sem).wait()

        def group(g, carry):
            uvec = uixv[pl.ds(g * L, L)]
            fetch_blocks(utabT_h, uvec & -TW)
            uph = uvec & (TW - 1)
            for d in range(D):
                dv = jnp.full((L,), d, dtype=jnp.int32)
                w_d = w_lo[d] if d < L else w_hi[d - L]
                stage[d, :] = plsc.load_gather(blk, [lane, dv, uph]) * w_d

            ivec = iixv[pl.ds(g * L, L)]
            fetch_blocks(itabT_h, ivec & -TW)
            iph = ivec & (TW - 1)
            acc = bvec
            for d in range(D):
                dv = jnp.full((L,), d, dtype=jnp.int32)
                acc = acc + stage[d, :] * plsc.load_gather(blk, [lane, dv, iph])
            outv[pl.ds(g * L, L)] = acc
            return carry

        lax.fori_loop(0, NG, group, 0)

        pltpu.sync_copy(outv, out_h.at[pl.ds(base, bpw)])

    return sc_kernel


@functools.lru_cache(maxsize=None)
def _build_tc(Bt, N):
    CH = 8 * TW
    nch = Bt // CH

    def tc_body(uidx_ref, iidx_ref, ublk_ref, iblk_ref, w_ref, b_ref, out_ref):
        g0 = pl.program_id(0)
        g1 = pl.program_id(1)
        r = g0 * CH + g1
        up = uidx_ref[r] & (TW - 1)
        ip = iidx_ref[r] & (TW - 1)
        lanes = lax.broadcasted_iota(jnp.int32, (D, TW), 1)
        um = jnp.where(lanes == up, 1.0, 0.0)
        im = jnp.where(lanes == ip, 1.0, 0.0)
        uvec = jnp.sum(ublk_ref[...] * um, axis=1, keepdims=True)
        ivec = jnp.sum(iblk_ref[...] * im, axis=1, keepdims=True)
        val = jnp.sum(uvec * ivec * w_ref[...]) + b_ref[0]
        sub = lax.broadcasted_iota(jnp.int32, (8, TW), 0)
        ln = lax.broadcasted_iota(jnp.int32, (8, TW), 1)
        hit = (sub == g1 // TW) & (ln == g1 % TW)
        out_ref[...] = jnp.where(hit, val, out_ref[...])

    grid_spec = pltpu.PrefetchScalarGridSpec(
        num_scalar_prefetch=2,
        grid=(nch, CH),
        in_specs=[
            pl.BlockSpec((D, TW), lambda g0, g1, u, i: (0, u[g0 * CH + g1] // TW)),
            pl.BlockSpec((D, TW), lambda g0, g1, u, i: (0, i[g0 * CH + g1] // TW)),
            pl.BlockSpec((D, 1), lambda g0, g1, u, i: (0, 0)),
            pl.BlockSpec((1,), lambda g0, g1, u, i: (0,)),
        ],
        out_specs=pl.BlockSpec((8, TW), lambda g0, g1, u, i: (g0, 0)),
    )
    return pl.pallas_call(
        tc_body,
        grid_spec=grid_spec,
        out_shape=jax.ShapeDtypeStruct((nch * 8, TW), jnp.float32),
    )


def kernel(user_idx, item_idx, user_table, item_table, W, b):
    B = user_idx.shape[0]
    N = user_table.shape[0]
    S = (B * SC_FRAC_NUM // SC_FRAC_DEN) // 512 * 512
    utabT = user_table.T
    itabT = item_table.T
    sc_out = _build_sc(S)(
        user_idx[:S], item_idx[:S], utabT, itabT,
        W.reshape(-1), jnp.broadcast_to(b, (L,)))
    tc_out = _build_tc(B - S, N)(
        user_idx[S:], item_idx[S:], utabT, itabT, W.reshape(D, 1), b)
    return jnp.concatenate([sc_out, tc_out.reshape(-1)])


# speedup vs baseline: 20.8428x; 20.8428x over previous
import functools

import jax
import jax.numpy as jnp
from jax import lax
from jax.experimental import pallas as pl
from jax.experimental.pallas import tpu as pltpu
from jax.experimental.pallas import tpu_sc as plsc

D = 32
L = 16
TW = 128


@functools.lru_cache(maxsize=None)
def _build(B):
    info = plsc.get_sparse_core_info()
    NC, NS = info.num_cores, info.num_subcores
    NW = NC * NS
    bpw = B // NW
    NG = bpw // L

    mesh = plsc.VectorSubcoreMesh(core_axis_name="c", subcore_axis_name="s")

    @functools.partial(
        pl.kernel,
        mesh=mesh,
        out_type=jax.ShapeDtypeStruct((B,), jnp.float32),
        compiler_params=pltpu.CompilerParams(
            needs_layout_passes=False, disable_bounds_checks=True),
        scratch_types=[
            pltpu.VMEM((bpw,), jnp.int32),
            pltpu.VMEM((bpw,), jnp.int32),
            pltpu.VMEM((L, D, TW), jnp.float32),
            pltpu.VMEM((D, L), jnp.float32),
            pltpu.VMEM((D,), jnp.float32),
            pltpu.VMEM((L,), jnp.float32),
            pltpu.VMEM((bpw,), jnp.float32),
            pltpu.SemaphoreType.DMA,
        ],
    )
    def sc_kernel(uidx_h, iidx_h, utabT_h, itabT_h, w_h, b_h, out_h,
                  uixv, iixv, blk, stage, wv, bv, outv, sem):
        wid = lax.axis_index("s") * NC + lax.axis_index("c")
        base = wid * bpw

        pltpu.sync_copy(uidx_h.at[pl.ds(base, bpw)], uixv)
        pltpu.sync_copy(iidx_h.at[pl.ds(base, bpw)], iixv)
        pltpu.sync_copy(w_h, wv)
        pltpu.sync_copy(b_h, bv)

        w_lo = wv[pl.ds(0, L)]
        w_hi = wv[pl.ds(L, L)]
        bvec = bv[...]
        lane = lax.iota(jnp.int32, L)
        zero = jnp.zeros((), jnp.int32)

        def fetch_item_blocks(cs):
            for j in range(L):
                off = pl.multiple_of(cs[j], TW)
                pltpu.async_copy(itabT_h.at[:, pl.ds(off, TW)], blk.at[j], sem)
            for j in range(L):
                pltpu.make_async_copy(
                    itabT_h.at[:, pl.ds(0, TW)], blk.at[j], sem).wait()

        def fetch_user_blocks(cs):
            slots = jnp.zeros((L,), jnp.int32)
            slot = zero
            nfetch = zero
            for j in range(L):
                if j == 0:
                    is_new = jnp.bool_(True)
                else:
                    is_new = cs[j] != cs[j - 1]
                slot = jnp.where(is_new, nfetch, slot)
                nfetch = nfetch + jnp.where(is_new, 1, 0)
                off = pl.multiple_of(cs[j], TW)

                @pl.when(is_new)
                def _(off=off, slot=slot):
                    pltpu.async_copy(
                        utabT_h.at[:, pl.ds(off, TW)], blk.at[slot], sem)

                slots = jnp.where(lane == j, slot, slots)

            def wait_one(k, carry):
                pltpu.make_async_copy(
                    utabT_h.at[:, pl.ds(0, TW)], blk.at[0], sem).wait()
                return carry

            lax.fori_loop(0, nfetch, wait_one, 0)
            return slots

        def group(g, carry):
            uvec = uixv[pl.ds(g * L, L)]
            slots = fetch_user_blocks(uvec & -TW)
            uph = uvec & (TW - 1)
            for d in range(D):
                dv = jnp.full((L,), d, dtype=jnp.int32)
                w_d = w_lo[d] if d < L else w_hi[d - L]
                stage[d, :] = plsc.load_gather(blk, [slots, dv, uph]) * w_d

            ivec = iixv[pl.ds(g * L, L)]
            fetch_item_blocks(ivec & -TW)
            iph = ivec & (TW - 1)
            acc = bvec
            for d in range(D):
                dv = jnp.full((L,), d, dtype=jnp.int32)
                acc = acc + stage[d, :] * plsc.load_gather(blk, [lane, dv, iph])
            outv[pl.ds(g * L, L)] = acc
            return carry

        lax.fori_loop(0, NG, group, 0)

        pltpu.sync_copy(outv, out_h.at[pl.ds(base, bpw)])

    return sc_kernel


def kernel(user_idx, item_idx, user_table, item_table, W, b):
    B = user_idx.shape[0]
    rows = lax.iota(jnp.int32, B)
    su, perm = lax.sort_key_val(user_idx, rows)
    si = jnp.take(item_idx, perm)
    out_sorted = _build(B)(
        su, si, user_table.T, item_table.T,
        W.reshape(-1), jnp.broadcast_to(b, (L,)))
    _, out = lax.sort_key_val(perm, out_sorted)
    return out
